# Initial kernel scaffold; baseline (speedup 1.0000x reference)
import jax
import jax.numpy as jnp
from jax.experimental import pallas as pl


def kernel(x, edge_index, W0, W1, b1, R1, rb1, W2, b2, R2, rb2, W3, b3, R3, rb3):
    raise NotImplementedError("write your pallas kernel here")


# speedup vs baseline: 2.7113x; 2.7113x over previous
import functools

import jax
import jax.numpy as jnp
from jax import lax
from jax.experimental import pallas as pl
from jax.experimental.pallas import tpu as pltpu
from jax.experimental.pallas import tpu_sc as plsc

N = 10000
E = 320000
DIM = 128
IN_FEATS = 75

NC = 2
NS = 16
NW = NC * NS
CH = 128
EPT = 10240
NCHUNK = EPT // CH
E_PAD = NW * EPT

N_ACC = 10112
PAD_ROW = 10008
ZROWS = N_ACC // NS
WROWS = 624
WROWS_LAST = N - 15 * WROWS

_sc_mesh = plsc.VectorSubcoreMesh(core_axis_name="c", subcore_axis_name="s",
                                  num_cores=NC, num_subcores=NS)


@functools.partial(
    pl.kernel,
    out_type=jax.ShapeDtypeStruct((NC, N, DIM), jnp.float32),
    mesh=_sc_mesh,
    scratch_types=[
        pltpu.VMEM((NCHUNK, CH), jnp.int32),
        pltpu.VMEM((NCHUNK, CH), jnp.int32),
        pltpu.VMEM((CH, DIM), jnp.float32),
        pltpu.VMEM_SHARED((N_ACC, DIM), jnp.float32),
        pltpu.SemaphoreType.DMA,
    ],
)
def _sc_aggregate(hw_hbm, src_hbm, dst_hbm, zeros_hbm, out_hbm,
                  src_v, dst_v, rows_v, acc, sem):
    cid = lax.axis_index("c")
    sid = lax.axis_index("s")
    wid = cid * NS + sid

    pltpu.sync_copy(src_hbm.at[wid], src_v)
    pltpu.sync_copy(dst_hbm.at[wid], dst_v)
    pltpu.sync_copy(zeros_hbm, acc.at[pl.ds(sid * ZROWS, ZROWS)])
    plsc.subcore_barrier()

    def chunk(c, carry):
        pltpu.async_copy(hw_hbm.at[src_v.at[c]], rows_v, sem).wait()
        pltpu.sync_copy(rows_v, acc.at[dst_v.at[c]], add=True)
        return carry

    lax.fori_loop(0, NCHUNK, chunk, 0, unroll=False)
    plsc.subcore_barrier()

    @pl.when(sid < NS - 1)
    def _():
        pltpu.sync_copy(acc.at[pl.ds(sid * WROWS, WROWS)],
                        out_hbm.at[cid, pl.ds(sid * WROWS, WROWS)])

    @pl.when(sid == NS - 1)
    def _():
        pltpu.sync_copy(acc.at[pl.ds(15 * WROWS, WROWS_LAST)],
                        out_hbm.at[cid, pl.ds(15 * WROWS, WROWS_LAST)])


ROWS = 2000


def _tc_entry_body(x_ref, w0t_ref, w1t_ref, r1t_ref, rb1_ref, hw_ref, res_ref):
    h = jnp.dot(x_ref[...], w0t_ref[...], preferred_element_type=jnp.float32)
    hw_ref[...] = jnp.dot(h, w1t_ref[...], preferred_element_type=jnp.float32)
    res_ref[...] = jnp.maximum(
        jnp.dot(h, r1t_ref[...], preferred_element_type=jnp.float32)
        + rb1_ref[...], 0.0)


def _tc_mid_body(agg_ref, b_ref, res_ref, wnt_ref, rnt_ref, rbn_ref,
                 hw_ref, resn_ref):
    h = jnp.maximum(agg_ref[0] + agg_ref[1] + b_ref[...], 0.0) + res_ref[...]
    hw_ref[...] = jnp.dot(h, wnt_ref[...], preferred_element_type=jnp.float32)
    resn_ref[...] = jnp.maximum(
        jnp.dot(h, rnt_ref[...], preferred_element_type=jnp.float32)
        + rbn_ref[...], 0.0)


def _tc_final_body(agg_ref, b_ref, res_ref, out_ref):
    out_ref[...] = (jnp.maximum(agg_ref[0] + agg_ref[1] + b_ref[...], 0.0)
                    + res_ref[...])


_row_spec = pl.BlockSpec((ROWS, DIM), lambda i: (i, 0))
_w_spec = pl.BlockSpec((DIM, DIM), lambda i: (0, 0))
_b_spec = pl.BlockSpec((1, DIM), lambda i: (0, 0))
_agg_spec = pl.BlockSpec((NC, ROWS, DIM), lambda i: (0, i, 0))
_f32 = jnp.float32

_tc_entry = pl.pallas_call(
    _tc_entry_body,
    grid=(N // ROWS,),
    in_specs=[_row_spec, _w_spec, _w_spec, _w_spec, _b_spec],
    out_specs=[_row_spec, _row_spec],
    out_shape=[jax.ShapeDtypeStruct((N, DIM), _f32)] * 2,
)

_tc_mid = pl.pallas_call(
    _tc_mid_body,
    grid=(N // ROWS,),
    in_specs=[_agg_spec, _b_spec, _row_spec, _w_spec, _w_spec, _b_spec],
    out_specs=[_row_spec, _row_spec],
    out_shape=[jax.ShapeDtypeStruct((N, DIM), _f32)] * 2,
)

_tc_final = pl.pallas_call(
    _tc_final_body,
    grid=(N // ROWS,),
    in_specs=[_agg_spec, _b_spec, _row_spec],
    out_specs=_row_spec,
    out_shape=jax.ShapeDtypeStruct((N, DIM), _f32),
)


def kernel(x, edge_index, W0, W1, b1, R1, rb1, W2, b2, R2, rb2, W3, b3, R3, rb3):
    x_pad = jnp.pad(x, ((0, 0), (0, DIM - IN_FEATS)))
    w0t = jnp.pad(W0.T, ((0, DIM - IN_FEATS), (0, 0)))
    pad = E_PAD - E
    src3 = jnp.concatenate(
        [edge_index[0], jnp.zeros((pad,), jnp.int32)]).reshape(NW, NCHUNK, CH)
    dst3 = jnp.concatenate(
        [edge_index[1], jnp.full((pad,), PAD_ROW, jnp.int32)]).reshape(
            NW, NCHUNK, CH)
    zeros = jnp.zeros((ZROWS, DIM), _f32)
    b1r, b2r, b3r = b1.reshape(1, DIM), b2.reshape(1, DIM), b3.reshape(1, DIM)
    rb1r, rb2r, rb3r = (rb1.reshape(1, DIM), rb2.reshape(1, DIM),
                        rb3.reshape(1, DIM))

    hw, res = _tc_entry(x_pad, w0t, W1.T, R1.T, rb1r)
    agg = _sc_aggregate(hw, src3, dst3, zeros)
    hw, res = _tc_mid(agg, b1r, res, W2.T, R2.T, rb2r)
    agg = _sc_aggregate(hw, src3, dst3, zeros)
    hw, res = _tc_mid(agg, b2r, res, W3.T, R3.T, rb3r)
    agg = _sc_aggregate(hw, src3, dst3, zeros)
    h = _tc_final(agg, b3r, res)
    return h.reshape(1, N, DIM)


# speedup vs baseline: 3.1081x; 1.1463x over previous
import functools

import jax
import jax.numpy as jnp
from jax import lax
from jax.experimental import pallas as pl
from jax.experimental.pallas import tpu as pltpu
from jax.experimental.pallas import tpu_sc as plsc

N = 10000
E = 320000
DIM = 128
IN_FEATS = 75

NC = 2
NS = 16
NW = NC * NS
CH = 128
EPT = 10240
NCHUNK = EPT // CH
E_PAD = NW * EPT

N_ACC = 10240
PAD_ROW = 10008
ZROWS = N_ACC // NS
WROWS = 624
WROWS_LAST = N - 15 * WROWS

_sc_mesh = plsc.VectorSubcoreMesh(core_axis_name="c", subcore_axis_name="s",
                                  num_cores=NC, num_subcores=NS)


NBUF = 2
PH = 2
CPP = NCHUNK // PH


@functools.partial(
    pl.kernel,
    out_type=jax.ShapeDtypeStruct((NC, N, DIM), jnp.float32),
    mesh=_sc_mesh,
    scratch_types=[
        pltpu.VMEM((CPP, CH), jnp.int32),
        pltpu.VMEM((CPP, CH), jnp.int32),
        pltpu.VMEM((NBUF, CH, DIM), jnp.float32),
        pltpu.VMEM_SHARED((N_ACC, DIM), jnp.float32),
        [pltpu.SemaphoreType.DMA] * NBUF,
    ],
)
def _sc_aggregate(hw_hbm, src_hbm, dst_hbm, zeros_hbm, out_hbm,
                  src_v, dst_v, rows_v, acc, gsem):
    cid = lax.axis_index("c")
    sid = lax.axis_index("s")
    wid = cid * NS + sid

    pltpu.sync_copy(zeros_hbm, rows_v.at[0])
    base = sid * ZROWS
    for z in range(ZROWS // CH):
        pltpu.sync_copy(rows_v.at[0], acc.at[pl.ds(base + z * CH, CH)])

    plsc.subcore_barrier()

    for p in range(PH):
        pltpu.sync_copy(src_hbm.at[wid, p], src_v)
        pltpu.sync_copy(dst_hbm.at[wid, p], dst_v)
        pltpu.async_copy(hw_hbm.at[src_v.at[0]], rows_v.at[0], gsem[0])

        def group(g, carry):
            for b in range(NBUF):
                c = g * NBUF + b
                pltpu.make_async_copy(
                    hw_hbm.at[pl.ds(0, CH)], rows_v.at[b], gsem[b]).wait()

                @pl.when(c < CPP - 1)
                def _():
                    pltpu.async_copy(hw_hbm.at[src_v.at[c + 1]],
                                     rows_v.at[1 - b], gsem[1 - b])

                pltpu.sync_copy(rows_v.at[b], acc.at[dst_v.at[c]], add=True)
            return carry

        lax.fori_loop(0, CPP // NBUF, group, 0, unroll=False)

    plsc.subcore_barrier()

    @pl.when(sid < NS - 1)
    def _():
        pltpu.sync_copy(acc.at[pl.ds(sid * WROWS, WROWS)],
                        out_hbm.at[cid, pl.ds(sid * WROWS, WROWS)])

    @pl.when(sid == NS - 1)
    def _():
        pltpu.sync_copy(acc.at[pl.ds(15 * WROWS, WROWS_LAST)],
                        out_hbm.at[cid, pl.ds(15 * WROWS, WROWS_LAST)])


ROWS = 2000


def _tc_entry_body(x_ref, w0t_ref, w1t_ref, r1t_ref, rb1_ref, hw_ref, res_ref):
    h = jnp.dot(x_ref[...], w0t_ref[...], preferred_element_type=jnp.float32)
    hw_ref[...] = jnp.dot(h, w1t_ref[...], preferred_element_type=jnp.float32)
    res_ref[...] = jnp.maximum(
        jnp.dot(h, r1t_ref[...], preferred_element_type=jnp.float32)
        + rb1_ref[...], 0.0)


def _tc_mid_body(agg_ref, b_ref, res_ref, wnt_ref, rnt_ref, rbn_ref,
                 hw_ref, resn_ref):
    h = jnp.maximum(agg_ref[0] + agg_ref[1] + b_ref[...], 0.0) + res_ref[...]
    hw_ref[...] = jnp.dot(h, wnt_ref[...], preferred_element_type=jnp.float32)
    resn_ref[...] = jnp.maximum(
        jnp.dot(h, rnt_ref[...], preferred_element_type=jnp.float32)
        + rbn_ref[...], 0.0)


def _tc_final_body(agg_ref, b_ref, res_ref, out_ref):
    out_ref[...] = (jnp.maximum(agg_ref[0] + agg_ref[1] + b_ref[...], 0.0)
                    + res_ref[...])


_row_spec = pl.BlockSpec((ROWS, DIM), lambda i: (i, 0))
_w_spec = pl.BlockSpec((DIM, DIM), lambda i: (0, 0))
_b_spec = pl.BlockSpec((1, DIM), lambda i: (0, 0))
_agg_spec = pl.BlockSpec((NC, ROWS, DIM), lambda i: (0, i, 0))
_f32 = jnp.float32

_tc_entry = pl.pallas_call(
    _tc_entry_body,
    grid=(N // ROWS,),
    in_specs=[_row_spec, _w_spec, _w_spec, _w_spec, _b_spec],
    out_specs=[_row_spec, _row_spec],
    out_shape=[jax.ShapeDtypeStruct((N, DIM), _f32)] * 2,
)

_tc_mid = pl.pallas_call(
    _tc_mid_body,
    grid=(N // ROWS,),
    in_specs=[_agg_spec, _b_spec, _row_spec, _w_spec, _w_spec, _b_spec],
    out_specs=[_row_spec, _row_spec],
    out_shape=[jax.ShapeDtypeStruct((N, DIM), _f32)] * 2,
)

_tc_final = pl.pallas_call(
    _tc_final_body,
    grid=(N // ROWS,),
    in_specs=[_agg_spec, _b_spec, _row_spec],
    out_specs=_row_spec,
    out_shape=jax.ShapeDtypeStruct((N, DIM), _f32),
)


def kernel(x, edge_index, W0, W1, b1, R1, rb1, W2, b2, R2, rb2, W3, b3, R3, rb3):
    x_pad = jnp.pad(x, ((0, 0), (0, DIM - IN_FEATS)))
    w0t = jnp.pad(W0.T, ((0, DIM - IN_FEATS), (0, 0)))
    pad = E_PAD - E
    src3 = jnp.concatenate(
        [edge_index[0], jnp.zeros((pad,), jnp.int32)]).reshape(NW, PH, CPP, CH)
    dst3 = jnp.concatenate(
        [edge_index[1], jnp.full((pad,), PAD_ROW, jnp.int32)]).reshape(
            NW, PH, CPP, CH)
    zeros = jnp.zeros((CH, DIM), _f32)
    b1r, b2r, b3r = b1.reshape(1, DIM), b2.reshape(1, DIM), b3.reshape(1, DIM)
    rb1r, rb2r, rb3r = (rb1.reshape(1, DIM), rb2.reshape(1, DIM),
                        rb3.reshape(1, DIM))

    hw, res = _tc_entry(x_pad, w0t, W1.T, R1.T, rb1r)
    agg = _sc_aggregate(hw, src3, dst3, zeros)
    hw, res = _tc_mid(agg, b1r, res, W2.T, R2.T, rb2r)
    agg = _sc_aggregate(hw, src3, dst3, zeros)
    hw, res = _tc_mid(agg, b2r, res, W3.T, R3.T, rb3r)
    agg = _sc_aggregate(hw, src3, dst3, zeros)
    h = _tc_final(agg, b3r, res)
    return h.reshape(1, N, DIM)


# speedup vs baseline: 12.0519x; 3.8776x over previous
import functools

import jax
import jax.numpy as jnp
from jax import lax
from jax.experimental import pallas as pl
from jax.experimental.pallas import tpu as pltpu
from jax.experimental.pallas import tpu_sc as plsc

N = 10000
E = 320000
DIM = 128
IN_FEATS = 75

NC = 2
NS = 16
NW = NC * NS
CH = 80
EPT = 10240
NCHUNK = EPT // CH
E_PAD = NW * EPT

N_ACC = 10240
ZROWS = N_ACC // NS
NJUNK = N_ACC - N
WROWS = 624
WROWS_LAST = N - 15 * WROWS

_sc_mesh = plsc.VectorSubcoreMesh(core_axis_name="c", subcore_axis_name="s",
                                  num_cores=NC, num_subcores=NS)


NBUF = 4
PH = 4
CPP = NCHUNK // PH


@functools.partial(
    pl.kernel,
    out_type=jax.ShapeDtypeStruct((NC, N, DIM), jnp.float32),
    mesh=_sc_mesh,
    scratch_types=[
        pltpu.VMEM((CPP, CH), jnp.int32),
        pltpu.VMEM((CPP, CH), jnp.int32),
        pltpu.VMEM((NBUF, CH, DIM), jnp.float32),
        pltpu.VMEM_SHARED((N_ACC, DIM), jnp.float32),
        [pltpu.SemaphoreType.DMA] * NBUF,
    ],
)
def _sc_aggregate(hw_hbm, src_hbm, dst_hbm, zeros_hbm, out_hbm,
                  src_v, dst_v, rows_v, acc, gsem):
    cid = lax.axis_index("c")
    sid = lax.axis_index("s")
    wid = cid * NS + sid

    pltpu.sync_copy(zeros_hbm, rows_v.at[0])
    base = sid * ZROWS
    for z in range(ZROWS // CH):
        pltpu.sync_copy(rows_v.at[0], acc.at[pl.ds(base + z * CH, CH)])

    plsc.subcore_barrier()

    for p in range(PH):
        pltpu.sync_copy(src_hbm.at[wid, p], src_v)
        pltpu.sync_copy(dst_hbm.at[wid, p], dst_v)
        for b in range(NBUF - 1):
            pltpu.async_copy(hw_hbm.at[src_v.at[b]], rows_v.at[b], gsem[b])

        def group(g, carry):
            for b in range(NBUF):
                c = g * NBUF + b
                pb = (b - 1) % NBUF
                pltpu.make_async_copy(
                    hw_hbm.at[pl.ds(0, CH)], rows_v.at[b], gsem[b]).wait()

                @pl.when(c <= CPP - NBUF)
                def _():
                    pltpu.async_copy(hw_hbm.at[src_v.at[c + NBUF - 1]],
                                     rows_v.at[pb], gsem[pb])

                pltpu.sync_copy(rows_v.at[b], acc.at[dst_v.at[c]], add=True)
            return carry

        lax.fori_loop(0, CPP // NBUF, group, 0, unroll=False)

    plsc.subcore_barrier()

    @pl.when(sid < NS - 1)
    def _():
        pltpu.sync_copy(acc.at[pl.ds(sid * WROWS, WROWS)],
                        out_hbm.at[cid, pl.ds(sid * WROWS, WROWS)])

    @pl.when(sid == NS - 1)
    def _():
        pltpu.sync_copy(acc.at[pl.ds(15 * WROWS, WROWS_LAST)],
                        out_hbm.at[cid, pl.ds(15 * WROWS, WROWS_LAST)])


ROWS = 2000


def _tc_entry_body(x_ref, w0t_ref, w1t_ref, r1t_ref, rb1_ref, hw_ref, res_ref):
    h = jnp.dot(x_ref[...], w0t_ref[...], preferred_element_type=jnp.float32)
    hw_ref[...] = jnp.dot(h, w1t_ref[...], preferred_element_type=jnp.float32)
    res_ref[...] = jnp.maximum(
        jnp.dot(h, r1t_ref[...], preferred_element_type=jnp.float32)
        + rb1_ref[...], 0.0)


def _tc_mid_body(agg_ref, b_ref, res_ref, wnt_ref, rnt_ref, rbn_ref,
                 hw_ref, resn_ref):
    h = jnp.maximum(agg_ref[0] + agg_ref[1] + b_ref[...], 0.0) + res_ref[...]
    hw_ref[...] = jnp.dot(h, wnt_ref[...], preferred_element_type=jnp.float32)
    resn_ref[...] = jnp.maximum(
        jnp.dot(h, rnt_ref[...], preferred_element_type=jnp.float32)
        + rbn_ref[...], 0.0)


def _tc_final_body(agg_ref, b_ref, res_ref, out_ref):
    out_ref[...] = (jnp.maximum(agg_ref[0] + agg_ref[1] + b_ref[...], 0.0)
                    + res_ref[...])


_row_spec = pl.BlockSpec((ROWS, DIM), lambda i: (i, 0))
_w_spec = pl.BlockSpec((DIM, DIM), lambda i: (0, 0))
_b_spec = pl.BlockSpec((1, DIM), lambda i: (0, 0))
_agg_spec = pl.BlockSpec((NC, ROWS, DIM), lambda i: (0, i, 0))
_f32 = jnp.float32

_tc_entry = pl.pallas_call(
    _tc_entry_body,
    grid=(N // ROWS,),
    in_specs=[_row_spec, _w_spec, _w_spec, _w_spec, _b_spec],
    out_specs=[_row_spec, _row_spec],
    out_shape=[jax.ShapeDtypeStruct((N, DIM), _f32)] * 2,
)

_tc_mid = pl.pallas_call(
    _tc_mid_body,
    grid=(N // ROWS,),
    in_specs=[_agg_spec, _b_spec, _row_spec, _w_spec, _w_spec, _b_spec],
    out_specs=[_row_spec, _row_spec],
    out_shape=[jax.ShapeDtypeStruct((N, DIM), _f32)] * 2,
)

_tc_final = pl.pallas_call(
    _tc_final_body,
    grid=(N // ROWS,),
    in_specs=[_agg_spec, _b_spec, _row_spec],
    out_specs=_row_spec,
    out_shape=jax.ShapeDtypeStruct((N, DIM), _f32),
)


def kernel(x, edge_index, W0, W1, b1, R1, rb1, W2, b2, R2, rb2, W3, b3, R3, rb3):
    x_pad = jnp.pad(x, ((0, 0), (0, DIM - IN_FEATS)))
    w0t = jnp.pad(W0.T, ((0, DIM - IN_FEATS), (0, 0)))
    pad = E_PAD - E
    pad_src = jnp.arange(pad, dtype=jnp.int32) % N
    src3 = jnp.concatenate([edge_index[0], pad_src]).reshape(NW, PH, CPP, CH)
    pad_dst = N + jnp.arange(pad, dtype=jnp.int32) % NJUNK
    dst3 = jnp.concatenate([edge_index[1], pad_dst]).reshape(NW, PH, CPP, CH)
    zeros = jnp.zeros((CH, DIM), _f32)
    b1r, b2r, b3r = b1.reshape(1, DIM), b2.reshape(1, DIM), b3.reshape(1, DIM)
    rb1r, rb2r, rb3r = (rb1.reshape(1, DIM), rb2.reshape(1, DIM),
                        rb3.reshape(1, DIM))

    hw, res = _tc_entry(x_pad, w0t, W1.T, R1.T, rb1r)
    agg = _sc_aggregate(hw, src3, dst3, zeros)
    hw, res = _tc_mid(agg, b1r, res, W2.T, R2.T, rb2r)
    agg = _sc_aggregate(hw, src3, dst3, zeros)
    hw, res = _tc_mid(agg, b2r, res, W3.T, R3.T, rb3r)
    agg = _sc_aggregate(hw, src3, dst3, zeros)
    h = _tc_final(agg, b3r, res)
    return h.reshape(1, N, DIM)
